# baseline (speedup 1.0000x reference)
import functools

import jax
import jax.numpy as jnp
from jax import lax
from jax.experimental import pallas as pl
from jax.experimental.pallas import tpu as pltpu
from jax.experimental.pallas import tpu_sc as plsc

N = 10000
E = 320000
D_EMB = 128
DH1 = 64
DH2 = 32

NC = 2
NS = 16
NW = NC * NS
CHUNK = 128
CPW = 80
TOTCH = NW * CPW
EP = TOTCH * CHUNK
NP = 10240
RPT = NP // NS
DUMMY = N

AGG64_SPLIT = (84, 76)
AGG32_SPLIT = (84, 76)
G2_SPLIT = (82, 78)

_mesh = plsc.VectorSubcoreMesh(core_axis_name="c", subcore_axis_name="s")
_sc_params = pltpu.CompilerParams(use_tc_tiling_on_sc=False)


@functools.partial(
    pl.kernel,
    out_type=jax.ShapeDtypeStruct((NC, NP, 8), jnp.float32),
    mesh=_mesh,
    compiler_params=_sc_params,
    scratch_types=[
        pltpu.VMEM((CPW, CHUNK), jnp.int32),
        pltpu.VMEM((CHUNK, 8), jnp.float32),
        pltpu.VMEM((RPT, 8), jnp.float32),
        pltpu.VMEM_SHARED((NP, 8), jnp.float32),
        pltpu.SemaphoreType.DMA,
    ],
)
def _deg_kernel(dstp, half, ones, out, idx_v, ones_v, bounce, table, sem):
    c = lax.axis_index("c")
    s = lax.axis_index("s")
    w = s * NC + c
    rows = pl.ds(s * RPT, RPT)
    pltpu.sync_copy(half.at[rows], bounce)
    pltpu.sync_copy(bounce, table.at[rows])
    pltpu.sync_copy(ones, ones_v)
    pltpu.sync_copy(dstp.at[w], idx_v)
    plsc.subcore_barrier()

    def body(j, carry):
        pltpu.sync_copy(ones_v, table.at[idx_v.at[j]], add=True)
        return carry

    lax.fori_loop(0, CPW, body, 0)
    plsc.subcore_barrier()
    pltpu.sync_copy(table.at[rows], bounce)
    pltpu.sync_copy(bounce, out.at[c, rows])


def _make_agg(D, split, spmem_gather=False, dtype=jnp.float32):
    c0, c1 = split
    cmax = max(c0, c1)
    scratch = [
        pltpu.VMEM((cmax, CHUNK), jnp.int32),
        pltpu.VMEM((cmax, CHUNK), jnp.int32),
        pltpu.VMEM((CHUNK, D), dtype),
        pltpu.VMEM((CHUNK, D), dtype),
        pltpu.VMEM((RPT, D), dtype),
        pltpu.VMEM_SHARED((NP, D), dtype),
        pltpu.SemaphoreType.DMA,
        pltpu.SemaphoreType.DMA,
    ]
    if spmem_gather:
        scratch.insert(5, pltpu.VMEM_SHARED((NP, D), dtype))

    @functools.partial(
        pl.kernel,
        out_type=jax.ShapeDtypeStruct((NC, NP, D), dtype),
        mesh=_mesh,
        compiler_params=_sc_params,
        scratch_types=scratch,
    )
    def agg(gtab_hbm, ghalf, srcq, dstq, out, idxs_v, idxd_v, rows_a, rows_b,
            bounce, *rest):
        if spmem_gather:
            gsh, table, sema, semb = rest
        else:
            table, sema, semb = rest
        c = lax.axis_index("c")
        s = lax.axis_index("s")
        rows = pl.ds(s * RPT, RPT)
        cnt = jnp.where(c == 0, c0, c1)
        pltpu.sync_copy(ghalf.at[rows], bounce)
        pltpu.sync_copy(bounce, table.at[rows])
        if spmem_gather:
            pltpu.sync_copy(gtab_hbm.at[rows], bounce)
            pltpu.sync_copy(bounce, gsh.at[rows])
            gtab = gsh
        else:
            gtab = gtab_hbm
        pltpu.sync_copy(srcq.at[c, s], idxs_v)
        pltpu.sync_copy(dstq.at[c, s], idxd_v)
        plsc.subcore_barrier()

        pltpu.async_copy(gtab.at[idxs_v.at[0]], rows_a, sema)

        def body(k, carry):
            j = 2 * k
            pltpu.make_async_copy(gtab.at[idxs_v.at[j]], rows_a, sema).wait()
            pltpu.async_copy(gtab.at[idxs_v.at[j + 1]], rows_b, semb)
            pltpu.sync_copy(rows_a, table.at[idxd_v.at[j]], add=True)
            pltpu.make_async_copy(gtab.at[idxs_v.at[j + 1]], rows_b, semb).wait()

            @pl.when(j + 2 < cnt)
            def _():
                pltpu.async_copy(gtab.at[idxs_v.at[j + 2]], rows_a, sema)

            pltpu.sync_copy(rows_b, table.at[idxd_v.at[j + 1]], add=True)
            return carry

        lax.fori_loop(0, cnt // 2, body, 0)
        plsc.subcore_barrier()
        pltpu.sync_copy(table.at[rows], bounce)
        pltpu.sync_copy(bounce, out.at[c, rows])

    return agg


_agg64 = _make_agg(DH1, AGG64_SPLIT, spmem_gather=True, dtype=jnp.bfloat16)
_agg32 = _make_agg(DH2, AGG32_SPLIT, spmem_gather=True)


_G2C0, _G2C1 = G2_SPLIT
_G2MAX = max(_G2C0, _G2C1)


@functools.partial(
    pl.kernel,
    out_type=jax.ShapeDtypeStruct((TOTCH * 2 * CHUNK, DH2), jnp.bfloat16),
    mesh=_mesh,
    compiler_params=_sc_params,
    scratch_types=[
        pltpu.VMEM((_G2MAX, CHUNK), jnp.int32),
        pltpu.VMEM((_G2MAX, CHUNK), jnp.int32),
        pltpu.VMEM((CHUNK, DH2), jnp.bfloat16),
        pltpu.VMEM((CHUNK, DH2), jnp.bfloat16),
        pltpu.VMEM((RPT, DH2), jnp.bfloat16),
        pltpu.VMEM_SHARED((NP, DH2), jnp.bfloat16),
        pltpu.SemaphoreType.DMA,
        pltpu.SemaphoreType.DMA,
    ],
)
def _gather2_kernel(zhbm, srcq, dstq, out, idx0_v, idx1_v, rows_a, rows_b,
                    bounce, z, sema, semb):
    c = lax.axis_index("c")
    s = lax.axis_index("s")
    rows = pl.ds(s * RPT, RPT)
    cnt = jnp.where(c == 0, _G2C0, _G2C1)
    goff = jnp.where(c == 0, s * _G2C0, NS * _G2C0 + s * _G2C1)
    pltpu.sync_copy(zhbm.at[rows], bounce)
    pltpu.sync_copy(bounce, z.at[rows])
    pltpu.sync_copy(srcq.at[c, s], idx0_v)
    pltpu.sync_copy(dstq.at[c, s], idx1_v)
    plsc.subcore_barrier()

    pltpu.async_copy(z.at[idx0_v.at[0]], rows_a, sema)

    def body(j, carry):
        g = goff + j
        pltpu.make_async_copy(z.at[idx0_v.at[j]], rows_a, sema).wait()
        pltpu.async_copy(z.at[idx1_v.at[j]], rows_b, semb)
        pltpu.sync_copy(rows_a, out.at[pl.ds(g * (2 * CHUNK), CHUNK)])
        pltpu.make_async_copy(z.at[idx1_v.at[j]], rows_b, semb).wait()

        @pl.when(j + 1 < cnt)
        def _():
            pltpu.async_copy(z.at[idx0_v.at[j + 1]], rows_a, sema)

        pltpu.sync_copy(rows_b, out.at[pl.ds(g * (2 * CHUNK) + CHUNK, CHUNK)])
        return carry

    lax.fori_loop(0, cnt, body, 0)


_BLK = 1024
_NB = NP // _BLK


def _tc1_body(degp, emb, w1, g1_o, gh_o, dinv_o):
    deg = degp[0] + degp[1]
    dinv = jnp.where(deg > 0, lax.rsqrt(jnp.maximum(deg, 1e-12)), 0.0)
    h = jnp.dot(emb[...], w1[...], preferred_element_type=jnp.float32)
    g = dinv[:, 0:1] * h
    g1_o[...] = g.astype(jnp.bfloat16)
    gh_o[...] = (0.5 * g).astype(jnp.bfloat16)
    dinv_o[...] = dinv


_tc1 = pl.pallas_call(
    _tc1_body,
    grid=(_NB,),
    in_specs=[
        pl.BlockSpec((NC, _BLK, 8), lambda i: (0, i, 0)),
        pl.BlockSpec((_BLK, D_EMB), lambda i: (i, 0)),
        pl.BlockSpec((D_EMB, DH1), lambda i: (0, 0)),
    ],
    out_specs=[
        pl.BlockSpec((_BLK, DH1), lambda i: (i, 0)),
        pl.BlockSpec((_BLK, DH1), lambda i: (i, 0)),
        pl.BlockSpec((_BLK, 8), lambda i: (i, 0)),
    ],
    out_shape=[
        jax.ShapeDtypeStruct((NP, DH1), jnp.bfloat16),
        jax.ShapeDtypeStruct((NP, DH1), jnp.bfloat16),
        jax.ShapeDtypeStruct((NP, 8), jnp.float32),
    ],
)


def _tc2_body(p, dinv, b1, w2, g2_o, gh_o):
    s = p[0].astype(jnp.float32) + p[1].astype(jnp.float32)
    out1 = dinv[:, 0:1] * s + b1[...]
    x1 = jnp.maximum(out1, 0.0)
    h2 = jnp.dot(x1, w2[...], preferred_element_type=jnp.float32)
    g = dinv[:, 0:1] * h2
    g2_o[...] = g
    gh_o[...] = 0.5 * g


_tc2 = pl.pallas_call(
    _tc2_body,
    grid=(_NB,),
    in_specs=[
        pl.BlockSpec((NC, _BLK, DH1), lambda i: (0, i, 0)),
        pl.BlockSpec((_BLK, 8), lambda i: (i, 0)),
        pl.BlockSpec((1, DH1), lambda i: (0, 0)),
        pl.BlockSpec((DH1, DH2), lambda i: (0, 0)),
    ],
    out_specs=[
        pl.BlockSpec((_BLK, DH2), lambda i: (i, 0)),
        pl.BlockSpec((_BLK, DH2), lambda i: (i, 0)),
    ],
    out_shape=[
        jax.ShapeDtypeStruct((NP, DH2), jnp.float32),
        jax.ShapeDtypeStruct((NP, DH2), jnp.float32),
    ],
)


def _tc3_body(q, dinv, b2, z_o):
    s = q[0] + q[1]
    z_o[...] = (dinv[:, 0:1] * s + b2[...]).astype(jnp.bfloat16)


_tc3 = pl.pallas_call(
    _tc3_body,
    grid=(_NB,),
    in_specs=[
        pl.BlockSpec((NC, _BLK, DH2), lambda i: (0, i, 0)),
        pl.BlockSpec((_BLK, 8), lambda i: (i, 0)),
        pl.BlockSpec((1, DH2), lambda i: (0, 0)),
    ],
    out_specs=pl.BlockSpec((_BLK, DH2), lambda i: (i, 0)),
    out_shape=jax.ShapeDtypeStruct((NP, DH2), jnp.bfloat16),
)


_GD = 64
_NDG = TOTCH // _GD


def _dec_body(a_ref, b_ref, p_o):
    a = a_ref[...].astype(jnp.float32)
    b = b_ref[...].astype(jnp.float32)
    ds = []
    for k in range(4):
        sl = slice(k * DH2, (k + 1) * DH2)
        ds.append(jnp.sum(a[:, 0, :, sl] * b[:, 0, :, sl], axis=2))
    d = jnp.stack(ds, axis=2).reshape(_GD, 128)
    p_o[...] = 1.0 / (1.0 + jnp.exp(-d))


_tc_dec = pl.pallas_call(
    _dec_body,
    grid=(_NDG,),
    in_specs=[
        pl.BlockSpec((_GD, 1, 32, 128), lambda i: (i, 0, 0, 0)),
        pl.BlockSpec((_GD, 1, 32, 128), lambda i: (i, 1, 0, 0)),
    ],
    out_specs=pl.BlockSpec((_GD, 128), lambda i: (i, 0)),
    out_shape=jax.ShapeDtypeStruct((TOTCH, 128), jnp.float32),
)


def _split_chunks(flat, c0, c1):
    cmax = max(c0, c1)
    a = flat[: NS * c0].reshape(NS, c0, CHUNK)
    b = flat[NS * c0:].reshape(NS, c1, CHUNK)
    a = jnp.pad(a, ((0, 0), (0, cmax - c0), (0, 0)))
    b = jnp.pad(b, ((0, 0), (0, cmax - c1), (0, 0)))
    return jnp.stack([a, b])


def kernel(edge_index, emb, W1, b1, W2, b2):
    src = edge_index[0]
    dst = edge_index[1]
    pad = EP - E
    dpad = N + (jnp.arange(pad, dtype=jnp.int32) % (NP - N))
    srcf = jnp.concatenate([src, jnp.zeros((pad,), jnp.int32)]).reshape(TOTCH, CHUNK)
    dstf = jnp.concatenate([dst, dpad]).reshape(TOTCH, CHUNK)
    dstp = dstf.reshape(NW, CPW, CHUNK)
    src64 = _split_chunks(srcf, *AGG64_SPLIT)
    dst64 = _split_chunks(dstf, *AGG64_SPLIT)
    src32 = _split_chunks(srcf, *AGG32_SPLIT)
    dst32 = _split_chunks(dstf, *AGG32_SPLIT)
    srcg2 = _split_chunks(srcf, *G2_SPLIT)
    dstg2 = _split_chunks(dstf, *G2_SPLIT)
    half8 = jnp.full((NP, 8), 0.5, jnp.float32)
    ones8 = jnp.ones((CHUNK, 8), jnp.float32)
    emb_p = jnp.pad(emb, ((0, NP - N), (0, 0)))

    degp = _deg_kernel(dstp, half8, ones8)
    g1, g1h, dinv8 = _tc1(degp, emb_p, W1)
    p1 = _agg64(g1, g1h, src64, dst64)
    g2, g2h = _tc2(p1, dinv8, b1.reshape(1, DH1), W2)
    p2 = _agg32(g2, g2h, src32, dst32)
    z = _tc3(p2, dinv8, b2.reshape(1, DH2))
    zz = _gather2_kernel(z, srcg2, dstg2)
    probs = _tc_dec(zz.reshape(TOTCH, 2, 32, 128), zz.reshape(TOTCH, 2, 32, 128))
    return probs.reshape(EP)[:E]


# speedup vs baseline: 1.3972x; 1.3972x over previous
import functools

import jax
import jax.numpy as jnp
from jax import lax
from jax.experimental import pallas as pl
from jax.experimental.pallas import tpu as pltpu
from jax.experimental.pallas import tpu_sc as plsc

N = 10000
E = 320000
D_EMB = 128
DH1 = 64
DH2 = 32

NC = 2
NS = 16
NW = NC * NS
CHUNK = 128
CPW = 80
TOTCH = NW * CPW
EP = TOTCH * CHUNK
NP = 10240
RPT = NP // NS
DUMMY = N

AGG64_SPLIT = (84, 76)
AGG32_SPLIT = (84, 76)
G2_SPLIT = (82, 78)

_mesh = plsc.VectorSubcoreMesh(core_axis_name="c", subcore_axis_name="s")
_sc_params = pltpu.CompilerParams(use_tc_tiling_on_sc=False)


@functools.partial(
    pl.kernel,
    out_type=jax.ShapeDtypeStruct((NC, NP, 8), jnp.float32),
    mesh=_mesh,
    compiler_params=_sc_params,
    scratch_types=[
        pltpu.VMEM((CPW, CHUNK), jnp.int32),
        pltpu.VMEM((CHUNK, 8), jnp.float32),
        pltpu.VMEM((RPT, 8), jnp.float32),
        pltpu.VMEM_SHARED((NP, 8), jnp.float32),
        pltpu.SemaphoreType.DMA,
    ],
)
def _deg_kernel(dstp, half, ones, out, idx_v, ones_v, bounce, table, sem):
    c = lax.axis_index("c")
    s = lax.axis_index("s")
    w = s * NC + c
    rows = pl.ds(s * RPT, RPT)
    pltpu.sync_copy(half.at[rows], bounce)
    pltpu.sync_copy(bounce, table.at[rows])
    pltpu.sync_copy(ones, ones_v)
    pltpu.sync_copy(dstp.at[w], idx_v)
    plsc.subcore_barrier()

    def body(j, carry):
        pltpu.sync_copy(ones_v, table.at[idx_v.at[j]], add=True)
        return carry

    lax.fori_loop(0, CPW, body, 0)
    plsc.subcore_barrier()
    pltpu.sync_copy(table.at[rows], bounce)
    pltpu.sync_copy(bounce, out.at[c, rows])


def _make_agg(D, split, spmem_gather=False, dtype=jnp.float32):
    c0, c1 = split
    cmax = max(c0, c1)
    scratch = [
        pltpu.VMEM((cmax, CHUNK), jnp.int32),
        pltpu.VMEM((cmax, CHUNK), jnp.int32),
        pltpu.VMEM((CHUNK, D), dtype),
        pltpu.VMEM((CHUNK, D), dtype),
        pltpu.VMEM((RPT, D), dtype),
        pltpu.VMEM_SHARED((NP, D), dtype),
        pltpu.SemaphoreType.DMA,
        pltpu.SemaphoreType.DMA,
    ]
    if spmem_gather:
        scratch.insert(5, pltpu.VMEM_SHARED((NP, D), dtype))

    @functools.partial(
        pl.kernel,
        out_type=jax.ShapeDtypeStruct((NC, NP, D), dtype),
        mesh=_mesh,
        compiler_params=_sc_params,
        scratch_types=scratch,
    )
    def agg(gtab_hbm, ghalf, srcq, dstq, out, idxs_v, idxd_v, rows_a, rows_b,
            bounce, *rest):
        if spmem_gather:
            gsh, table, sema, semb = rest
        else:
            table, sema, semb = rest
        c = lax.axis_index("c")
        s = lax.axis_index("s")
        rows = pl.ds(s * RPT, RPT)
        cnt = jnp.where(c == 0, c0, c1)
        pltpu.sync_copy(ghalf.at[rows], bounce)
        pltpu.sync_copy(bounce, table.at[rows])
        if spmem_gather:
            pltpu.sync_copy(gtab_hbm.at[rows], bounce)
            pltpu.sync_copy(bounce, gsh.at[rows])
            gtab = gsh
        else:
            gtab = gtab_hbm
        pltpu.sync_copy(srcq.at[c, s], idxs_v)
        pltpu.sync_copy(dstq.at[c, s], idxd_v)
        plsc.subcore_barrier()

        pltpu.async_copy(gtab.at[idxs_v.at[0]], rows_a, sema)

        def body(k, carry):
            j = 2 * k
            pltpu.make_async_copy(gtab.at[idxs_v.at[j]], rows_a, sema).wait()
            pltpu.async_copy(gtab.at[idxs_v.at[j + 1]], rows_b, semb)
            pltpu.sync_copy(rows_a, table.at[idxd_v.at[j]], add=True)
            pltpu.make_async_copy(gtab.at[idxs_v.at[j + 1]], rows_b, semb).wait()

            @pl.when(j + 2 < cnt)
            def _():
                pltpu.async_copy(gtab.at[idxs_v.at[j + 2]], rows_a, sema)

            pltpu.sync_copy(rows_b, table.at[idxd_v.at[j + 1]], add=True)
            return carry

        lax.fori_loop(0, cnt // 2, body, 0)
        plsc.subcore_barrier()
        pltpu.sync_copy(table.at[rows], bounce)
        pltpu.sync_copy(bounce, out.at[c, rows])

    return agg


_agg64 = _make_agg(DH1, AGG64_SPLIT, spmem_gather=True, dtype=jnp.bfloat16)
_agg32 = _make_agg(DH2, AGG32_SPLIT, spmem_gather=True)


_G2C0, _G2C1 = G2_SPLIT
_G2MAX = max(_G2C0, _G2C1)


@functools.partial(
    pl.kernel,
    out_type=jax.ShapeDtypeStruct((TOTCH * 2 * CHUNK, DH2), jnp.float32),
    mesh=_mesh,
    compiler_params=_sc_params,
    scratch_types=[
        pltpu.VMEM((_G2MAX, CHUNK), jnp.int32),
        pltpu.VMEM((_G2MAX, CHUNK), jnp.int32),
        pltpu.VMEM((CHUNK, DH2), jnp.float32),
        pltpu.VMEM((CHUNK, DH2), jnp.float32),
        pltpu.VMEM((RPT, DH2), jnp.float32),
        pltpu.VMEM_SHARED((NP, DH2), jnp.float32),
        pltpu.SemaphoreType.DMA,
        pltpu.SemaphoreType.DMA,
    ],
)
def _gather2_kernel(zhbm, srcq, dstq, out, idx0_v, idx1_v, rows_a, rows_b,
                    bounce, z, sema, semb):
    c = lax.axis_index("c")
    s = lax.axis_index("s")
    rows = pl.ds(s * RPT, RPT)
    cnt = jnp.where(c == 0, _G2C0, _G2C1)
    goff = jnp.where(c == 0, s * _G2C0, NS * _G2C0 + s * _G2C1)
    pltpu.sync_copy(zhbm.at[rows], bounce)
    pltpu.sync_copy(bounce, z.at[rows])
    pltpu.sync_copy(srcq.at[c, s], idx0_v)
    pltpu.sync_copy(dstq.at[c, s], idx1_v)
    plsc.subcore_barrier()

    pltpu.async_copy(z.at[idx0_v.at[0]], rows_a, sema)

    def body(j, carry):
        g = goff + j
        pltpu.make_async_copy(z.at[idx0_v.at[j]], rows_a, sema).wait()
        pltpu.async_copy(z.at[idx1_v.at[j]], rows_b, semb)
        pltpu.sync_copy(rows_a, out.at[pl.ds(g * (2 * CHUNK), CHUNK)])
        pltpu.make_async_copy(z.at[idx1_v.at[j]], rows_b, semb).wait()

        @pl.when(j + 1 < cnt)
        def _():
            pltpu.async_copy(z.at[idx0_v.at[j + 1]], rows_a, sema)

        pltpu.sync_copy(rows_b, out.at[pl.ds(g * (2 * CHUNK) + CHUNK, CHUNK)])
        return carry

    lax.fori_loop(0, cnt, body, 0)


_BLK = 1024
_NB = NP // _BLK


def _tc1_body(degp, emb, w1, g1_o, gh_o, dinv_o):
    deg = degp[0] + degp[1]
    dinv = jnp.where(deg > 0, lax.rsqrt(jnp.maximum(deg, 1e-12)), 0.0)
    h = jnp.dot(emb[...], w1[...], preferred_element_type=jnp.float32)
    g = dinv[:, 0:1] * h
    g1_o[...] = g.astype(jnp.bfloat16)
    gh_o[...] = (0.5 * g).astype(jnp.bfloat16)
    dinv_o[...] = dinv


_tc1 = pl.pallas_call(
    _tc1_body,
    grid=(_NB,),
    in_specs=[
        pl.BlockSpec((NC, _BLK, 8), lambda i: (0, i, 0)),
        pl.BlockSpec((_BLK, D_EMB), lambda i: (i, 0)),
        pl.BlockSpec((D_EMB, DH1), lambda i: (0, 0)),
    ],
    out_specs=[
        pl.BlockSpec((_BLK, DH1), lambda i: (i, 0)),
        pl.BlockSpec((_BLK, DH1), lambda i: (i, 0)),
        pl.BlockSpec((_BLK, 8), lambda i: (i, 0)),
    ],
    out_shape=[
        jax.ShapeDtypeStruct((NP, DH1), jnp.bfloat16),
        jax.ShapeDtypeStruct((NP, DH1), jnp.bfloat16),
        jax.ShapeDtypeStruct((NP, 8), jnp.float32),
    ],
)


def _tc2_body(p, dinv, b1, w2, g2_o, gh_o):
    s = p[0].astype(jnp.float32) + p[1].astype(jnp.float32)
    out1 = dinv[:, 0:1] * s + b1[...]
    x1 = jnp.maximum(out1, 0.0)
    h2 = jnp.dot(x1, w2[...], preferred_element_type=jnp.float32)
    g = dinv[:, 0:1] * h2
    g2_o[...] = g
    gh_o[...] = 0.5 * g


_tc2 = pl.pallas_call(
    _tc2_body,
    grid=(_NB,),
    in_specs=[
        pl.BlockSpec((NC, _BLK, DH1), lambda i: (0, i, 0)),
        pl.BlockSpec((_BLK, 8), lambda i: (i, 0)),
        pl.BlockSpec((1, DH1), lambda i: (0, 0)),
        pl.BlockSpec((DH1, DH2), lambda i: (0, 0)),
    ],
    out_specs=[
        pl.BlockSpec((_BLK, DH2), lambda i: (i, 0)),
        pl.BlockSpec((_BLK, DH2), lambda i: (i, 0)),
    ],
    out_shape=[
        jax.ShapeDtypeStruct((NP, DH2), jnp.float32),
        jax.ShapeDtypeStruct((NP, DH2), jnp.float32),
    ],
)


def _tc3_body(q, dinv, b2, z_o):
    s = q[0] + q[1]
    z_o[...] = dinv[:, 0:1] * s + b2[...]


_tc3 = pl.pallas_call(
    _tc3_body,
    grid=(_NB,),
    in_specs=[
        pl.BlockSpec((NC, _BLK, DH2), lambda i: (0, i, 0)),
        pl.BlockSpec((_BLK, 8), lambda i: (i, 0)),
        pl.BlockSpec((1, DH2), lambda i: (0, 0)),
    ],
    out_specs=pl.BlockSpec((_BLK, DH2), lambda i: (i, 0)),
    out_shape=jax.ShapeDtypeStruct((NP, DH2), jnp.float32),
)


_GD = 64
_NDG = TOTCH // _GD


def _dec_body(a_ref, b_ref, p_o):
    a = a_ref[...]
    b = b_ref[...]
    ds = []
    for k in range(4):
        sl = slice(k * DH2, (k + 1) * DH2)
        ds.append(jnp.sum(a[:, 0, :, sl] * b[:, 0, :, sl], axis=2))
    d = jnp.stack(ds, axis=2).reshape(_GD, 128)
    p_o[...] = 1.0 / (1.0 + jnp.exp(-d))


_tc_dec = pl.pallas_call(
    _dec_body,
    grid=(_NDG,),
    in_specs=[
        pl.BlockSpec((_GD, 1, 32, 128), lambda i: (i, 0, 0, 0)),
        pl.BlockSpec((_GD, 1, 32, 128), lambda i: (i, 1, 0, 0)),
    ],
    out_specs=pl.BlockSpec((_GD, 128), lambda i: (i, 0)),
    out_shape=jax.ShapeDtypeStruct((TOTCH, 128), jnp.float32),
)


def _split_chunks(flat, c0, c1):
    cmax = max(c0, c1)
    a = flat[: NS * c0].reshape(NS, c0, CHUNK)
    b = flat[NS * c0:].reshape(NS, c1, CHUNK)
    a = jnp.pad(a, ((0, 0), (0, cmax - c0), (0, 0)))
    b = jnp.pad(b, ((0, 0), (0, cmax - c1), (0, 0)))
    return jnp.stack([a, b])


def kernel(edge_index, emb, W1, b1, W2, b2):
    src = edge_index[0]
    dst = edge_index[1]
    pad = EP - E
    dpad = N + (jnp.arange(pad, dtype=jnp.int32) % (NP - N))
    srcf = jnp.concatenate([src, jnp.zeros((pad,), jnp.int32)]).reshape(TOTCH, CHUNK)
    dstf = jnp.concatenate([dst, dpad]).reshape(TOTCH, CHUNK)
    dstp = dstf.reshape(NW, CPW, CHUNK)
    src64 = _split_chunks(srcf, *AGG64_SPLIT)
    dst64 = _split_chunks(dstf, *AGG64_SPLIT)
    src32 = _split_chunks(srcf, *AGG32_SPLIT)
    dst32 = _split_chunks(dstf, *AGG32_SPLIT)
    srcg2 = _split_chunks(srcf, *G2_SPLIT)
    dstg2 = _split_chunks(dstf, *G2_SPLIT)
    half8 = jnp.full((NP, 8), 0.5, jnp.float32)
    ones8 = jnp.ones((CHUNK, 8), jnp.float32)
    emb_p = jnp.pad(emb, ((0, NP - N), (0, 0)))

    degp = _deg_kernel(dstp, half8, ones8)
    g1, g1h, dinv8 = _tc1(degp, emb_p, W1)
    p1 = _agg64(g1, g1h, src64, dst64)
    g2, g2h = _tc2(p1, dinv8, b1.reshape(1, DH1), W2)
    p2 = _agg32(g2, g2h, src32, dst32)
    z = _tc3(p2, dinv8, b2.reshape(1, DH2))
    zz = _gather2_kernel(z, srcg2, dstg2)
    probs = _tc_dec(zz.reshape(TOTCH, 2, 32, 128), zz.reshape(TOTCH, 2, 32, 128))
    return probs.reshape(EP)[:E]


# speedup vs baseline: 1.4497x; 1.0375x over previous
import functools

import jax
import jax.numpy as jnp
from jax import lax
from jax.experimental import pallas as pl
from jax.experimental.pallas import tpu as pltpu
from jax.experimental.pallas import tpu_sc as plsc

N = 10000
E = 320000
D_EMB = 128
DH1 = 64
DH2 = 32

NC = 2
NS = 16
NW = NC * NS
CHUNK = 128
CPW = 80
TOTCH = NW * CPW
EP = TOTCH * CHUNK
NP = 10240
RPT = NP // NS
DUMMY = N

AGG64_SPLIT = (88, 72)
AGG32_SPLIT = (88, 72)
G2_SPLIT = (84, 76)

_mesh = plsc.VectorSubcoreMesh(core_axis_name="c", subcore_axis_name="s")
_sc_params = pltpu.CompilerParams(use_tc_tiling_on_sc=False)


@functools.partial(
    pl.kernel,
    out_type=jax.ShapeDtypeStruct((NC, NP, 8), jnp.float32),
    mesh=_mesh,
    compiler_params=_sc_params,
    scratch_types=[
        pltpu.VMEM((CPW, CHUNK), jnp.int32),
        pltpu.VMEM((CHUNK, 8), jnp.float32),
        pltpu.VMEM((RPT, 8), jnp.float32),
        pltpu.VMEM_SHARED((NP, 8), jnp.float32),
        pltpu.SemaphoreType.DMA,
    ],
)
def _deg_kernel(dstp, half, ones, out, idx_v, ones_v, bounce, table, sem):
    c = lax.axis_index("c")
    s = lax.axis_index("s")
    w = s * NC + c
    rows = pl.ds(s * RPT, RPT)
    pltpu.sync_copy(half.at[rows], bounce)
    pltpu.sync_copy(bounce, table.at[rows])
    pltpu.sync_copy(ones, ones_v)
    pltpu.sync_copy(dstp.at[w], idx_v)
    plsc.subcore_barrier()

    def body(j, carry):
        pltpu.sync_copy(ones_v, table.at[idx_v.at[j]], add=True)
        return carry

    lax.fori_loop(0, CPW, body, 0)
    plsc.subcore_barrier()
    pltpu.sync_copy(table.at[rows], bounce)
    pltpu.sync_copy(bounce, out.at[c, rows])


def _make_agg(D, split, spmem_gather=False, dtype=jnp.float32):
    c0, c1 = split
    cmax = max(c0, c1)
    scratch = [
        pltpu.VMEM((cmax, CHUNK), jnp.int32),
        pltpu.VMEM((cmax, CHUNK), jnp.int32),
        pltpu.VMEM((CHUNK, D), dtype),
        pltpu.VMEM((CHUNK, D), dtype),
        pltpu.VMEM((RPT, D), dtype),
        pltpu.VMEM_SHARED((NP, D), dtype),
        pltpu.SemaphoreType.DMA,
        pltpu.SemaphoreType.DMA,
    ]
    if spmem_gather:
        scratch.insert(5, pltpu.VMEM_SHARED((NP, D), dtype))

    @functools.partial(
        pl.kernel,
        out_type=jax.ShapeDtypeStruct((NC, NP, D), dtype),
        mesh=_mesh,
        compiler_params=_sc_params,
        scratch_types=scratch,
    )
    def agg(gtab_hbm, ghalf, srcq, dstq, out, idxs_v, idxd_v, rows_a, rows_b,
            bounce, *rest):
        if spmem_gather:
            gsh, table, sema, semb = rest
        else:
            table, sema, semb = rest
        c = lax.axis_index("c")
        s = lax.axis_index("s")
        rows = pl.ds(s * RPT, RPT)
        cnt = jnp.where(c == 0, c0, c1)
        pltpu.sync_copy(ghalf.at[rows], bounce)
        pltpu.sync_copy(bounce, table.at[rows])
        if spmem_gather:
            pltpu.sync_copy(gtab_hbm.at[rows], bounce)
            pltpu.sync_copy(bounce, gsh.at[rows])
            gtab = gsh
        else:
            gtab = gtab_hbm
        pltpu.sync_copy(srcq.at[c, s], idxs_v)
        pltpu.sync_copy(dstq.at[c, s], idxd_v)
        plsc.subcore_barrier()

        pltpu.async_copy(gtab.at[idxs_v.at[0]], rows_a, sema)

        def body(k, carry):
            j = 2 * k
            pltpu.make_async_copy(gtab.at[idxs_v.at[j]], rows_a, sema).wait()
            pltpu.async_copy(gtab.at[idxs_v.at[j + 1]], rows_b, semb)
            pltpu.sync_copy(rows_a, table.at[idxd_v.at[j]], add=True)
            pltpu.make_async_copy(gtab.at[idxs_v.at[j + 1]], rows_b, semb).wait()

            @pl.when(j + 2 < cnt)
            def _():
                pltpu.async_copy(gtab.at[idxs_v.at[j + 2]], rows_a, sema)

            pltpu.sync_copy(rows_b, table.at[idxd_v.at[j + 1]], add=True)
            return carry

        lax.fori_loop(0, cnt // 2, body, 0)
        plsc.subcore_barrier()
        pltpu.sync_copy(table.at[rows], bounce)
        pltpu.sync_copy(bounce, out.at[c, rows])

    return agg


_agg64 = _make_agg(DH1, AGG64_SPLIT, spmem_gather=True, dtype=jnp.bfloat16)
_agg32 = _make_agg(DH2, AGG32_SPLIT, spmem_gather=True)


_G2C0, _G2C1 = G2_SPLIT
_G2MAX = max(_G2C0, _G2C1)


@functools.partial(
    pl.kernel,
    out_type=jax.ShapeDtypeStruct((TOTCH * 2 * CHUNK, DH2), jnp.float32),
    mesh=_mesh,
    compiler_params=_sc_params,
    scratch_types=[
        pltpu.VMEM((_G2MAX, CHUNK), jnp.int32),
        pltpu.VMEM((_G2MAX, CHUNK), jnp.int32),
        pltpu.VMEM((CHUNK, DH2), jnp.float32),
        pltpu.VMEM((CHUNK, DH2), jnp.float32),
        pltpu.VMEM((RPT, DH2), jnp.float32),
        pltpu.VMEM_SHARED((NP, DH2), jnp.float32),
        pltpu.SemaphoreType.DMA,
        pltpu.SemaphoreType.DMA,
    ],
)
def _gather2_kernel(zhbm, srcq, dstq, out, idx0_v, idx1_v, rows_a, rows_b,
                    bounce, z, sema, semb):
    c = lax.axis_index("c")
    s = lax.axis_index("s")
    rows = pl.ds(s * RPT, RPT)
    cnt = jnp.where(c == 0, _G2C0, _G2C1)
    goff = jnp.where(c == 0, s * _G2C0, NS * _G2C0 + s * _G2C1)
    pltpu.sync_copy(zhbm.at[rows], bounce)
    pltpu.sync_copy(bounce, z.at[rows])
    pltpu.sync_copy(srcq.at[c, s], idx0_v)
    pltpu.sync_copy(dstq.at[c, s], idx1_v)
    plsc.subcore_barrier()

    pltpu.async_copy(z.at[idx0_v.at[0]], rows_a, sema)

    def body(j, carry):
        g = goff + j
        pltpu.make_async_copy(z.at[idx0_v.at[j]], rows_a, sema).wait()
        pltpu.async_copy(z.at[idx1_v.at[j]], rows_b, semb)
        pltpu.sync_copy(rows_a, out.at[pl.ds(g * (2 * CHUNK), CHUNK)])
        pltpu.make_async_copy(z.at[idx1_v.at[j]], rows_b, semb).wait()

        @pl.when(j + 1 < cnt)
        def _():
            pltpu.async_copy(z.at[idx0_v.at[j + 1]], rows_a, sema)

        pltpu.sync_copy(rows_b, out.at[pl.ds(g * (2 * CHUNK) + CHUNK, CHUNK)])
        return carry

    lax.fori_loop(0, cnt, body, 0)


_BLK = 1024
_NB = NP // _BLK


def _tc1_body(degp, emb, w1, g1_o, dinv_o):
    deg = degp[0] + degp[1] + 1.0
    dinv = jnp.where(deg > 0, lax.rsqrt(jnp.maximum(deg, 1e-12)), 0.0)
    h = jnp.dot(emb[...], w1[...], preferred_element_type=jnp.float32)
    g = dinv[:, 0:1] * h
    g1_o[...] = g.astype(jnp.bfloat16)
    dinv_o[...] = dinv


_tc1 = pl.pallas_call(
    _tc1_body,
    grid=(_NB,),
    in_specs=[
        pl.BlockSpec((NC, _BLK, 8), lambda i: (0, i, 0)),
        pl.BlockSpec((_BLK, D_EMB), lambda i: (i, 0)),
        pl.BlockSpec((D_EMB, DH1), lambda i: (0, 0)),
    ],
    out_specs=[
        pl.BlockSpec((_BLK, DH1), lambda i: (i, 0)),
        pl.BlockSpec((_BLK, 8), lambda i: (i, 0)),
    ],
    out_shape=[
        jax.ShapeDtypeStruct((NP, DH1), jnp.bfloat16),
        jax.ShapeDtypeStruct((NP, 8), jnp.float32),
    ],
)


def _tc2_body(p, g1, dinv, b1, w2, g2_o):
    s = (p[0].astype(jnp.float32) + p[1].astype(jnp.float32)
         + g1[...].astype(jnp.float32))
    out1 = dinv[:, 0:1] * s + b1[...]
    x1 = jnp.maximum(out1, 0.0)
    h2 = jnp.dot(x1, w2[...], preferred_element_type=jnp.float32)
    g2_o[...] = dinv[:, 0:1] * h2


_tc2 = pl.pallas_call(
    _tc2_body,
    grid=(_NB,),
    in_specs=[
        pl.BlockSpec((NC, _BLK, DH1), lambda i: (0, i, 0)),
        pl.BlockSpec((_BLK, DH1), lambda i: (i, 0)),
        pl.BlockSpec((_BLK, 8), lambda i: (i, 0)),
        pl.BlockSpec((1, DH1), lambda i: (0, 0)),
        pl.BlockSpec((DH1, DH2), lambda i: (0, 0)),
    ],
    out_specs=pl.BlockSpec((_BLK, DH2), lambda i: (i, 0)),
    out_shape=jax.ShapeDtypeStruct((NP, DH2), jnp.float32),
)


def _tc3_body(q, g2, dinv, b2, z_o):
    s = q[0] + q[1] + g2[...]
    z_o[...] = dinv[:, 0:1] * s + b2[...]


_tc3 = pl.pallas_call(
    _tc3_body,
    grid=(_NB,),
    in_specs=[
        pl.BlockSpec((NC, _BLK, DH2), lambda i: (0, i, 0)),
        pl.BlockSpec((_BLK, DH2), lambda i: (i, 0)),
        pl.BlockSpec((_BLK, 8), lambda i: (i, 0)),
        pl.BlockSpec((1, DH2), lambda i: (0, 0)),
    ],
    out_specs=pl.BlockSpec((_BLK, DH2), lambda i: (i, 0)),
    out_shape=jax.ShapeDtypeStruct((NP, DH2), jnp.float32),
)


_GD = 64
_NDG = TOTCH // _GD


def _dec_body(a_ref, b_ref, p_o):
    a = a_ref[...]
    b = b_ref[...]
    ds = []
    for k in range(4):
        sl = slice(k * DH2, (k + 1) * DH2)
        ds.append(jnp.sum(a[:, 0, :, sl] * b[:, 0, :, sl], axis=2))
    d = jnp.stack(ds, axis=2).reshape(_GD, 128)
    p_o[...] = 1.0 / (1.0 + jnp.exp(-d))


_tc_dec = pl.pallas_call(
    _dec_body,
    grid=(_NDG,),
    in_specs=[
        pl.BlockSpec((_GD, 1, 32, 128), lambda i: (i, 0, 0, 0)),
        pl.BlockSpec((_GD, 1, 32, 128), lambda i: (i, 1, 0, 0)),
    ],
    out_specs=pl.BlockSpec((_GD, 128), lambda i: (i, 0)),
    out_shape=jax.ShapeDtypeStruct((TOTCH, 128), jnp.float32),
)


def _split_chunks(flat, c0, c1):
    cmax = max(c0, c1)
    a = flat[: NS * c0].reshape(NS, c0, CHUNK)
    b = flat[NS * c0:].reshape(NS, c1, CHUNK)
    a = jnp.pad(a, ((0, 0), (0, cmax - c0), (0, 0)))
    b = jnp.pad(b, ((0, 0), (0, cmax - c1), (0, 0)))
    return jnp.stack([a, b])


def kernel(edge_index, emb, W1, b1, W2, b2):
    src = edge_index[0]
    dst = edge_index[1]
    pad = EP - E
    dpad = N + (jnp.arange(pad, dtype=jnp.int32) % (NP - N))
    srcf = jnp.concatenate([src, jnp.zeros((pad,), jnp.int32)]).reshape(TOTCH, CHUNK)
    dstf = jnp.concatenate([dst, dpad]).reshape(TOTCH, CHUNK)
    dstp = dstf.reshape(NW, CPW, CHUNK)
    src64 = _split_chunks(srcf, *AGG64_SPLIT)
    dst64 = _split_chunks(dstf, *AGG64_SPLIT)
    src32 = _split_chunks(srcf, *AGG32_SPLIT)
    dst32 = _split_chunks(dstf, *AGG32_SPLIT)
    srcg2 = _split_chunks(srcf, *G2_SPLIT)
    dstg2 = _split_chunks(dstf, *G2_SPLIT)
    zero8 = jnp.zeros((NP, 8), jnp.float32)
    zero64 = jnp.zeros((NP, DH1), jnp.bfloat16)
    zero32 = jnp.zeros((NP, DH2), jnp.float32)
    ones8 = jnp.ones((CHUNK, 8), jnp.float32)
    emb_p = jnp.pad(emb, ((0, NP - N), (0, 0)))

    degp = _deg_kernel(dstp, zero8, ones8)
    g1, dinv8 = _tc1(degp, emb_p, W1)
    p1 = _agg64(g1, zero64, src64, dst64)
    g2 = _tc2(p1, g1, dinv8, b1.reshape(1, DH1), W2)
    p2 = _agg32(g2, zero32, src32, dst32)
    z = _tc3(p2, g2, dinv8, b2.reshape(1, DH2))
    zz = _gather2_kernel(z, srcg2, dstg2)
    probs = _tc_dec(zz.reshape(TOTCH, 2, 32, 128), zz.reshape(TOTCH, 2, 32, 128))
    return probs.reshape(EP)[:E]


# speedup vs baseline: 1.5455x; 1.0661x over previous
import functools

import jax
import jax.numpy as jnp
from jax import lax
from jax.experimental import pallas as pl
from jax.experimental.pallas import tpu as pltpu
from jax.experimental.pallas import tpu_sc as plsc

N = 10000
E = 320000
D_EMB = 128
DH1 = 64
DH2 = 32

NC = 2
NS = 16
NW = NC * NS
CHUNK = 128
CPW = 80
TOTCH = NW * CPW
EP = TOTCH * CHUNK
NP = 10240
RPT = NP // NS
DUMMY = N

AGG64_SPLIT = (88, 72)
AGG32_SPLIT = (88, 72)
G2_SPLIT = (84, 76)

_mesh = plsc.VectorSubcoreMesh(core_axis_name="c", subcore_axis_name="s")
_sc_params = pltpu.CompilerParams(use_tc_tiling_on_sc=False)


@functools.partial(
    pl.kernel,
    out_type=jax.ShapeDtypeStruct((NC, NP, 8), jnp.float32),
    mesh=_mesh,
    compiler_params=_sc_params,
    scratch_types=[
        pltpu.VMEM((CPW, CHUNK), jnp.int32),
        pltpu.VMEM((CHUNK, 8), jnp.float32),
        pltpu.VMEM((RPT, 8), jnp.float32),
        pltpu.VMEM_SHARED((NP, 8), jnp.float32),
        pltpu.SemaphoreType.DMA,
    ],
)
def _deg_kernel(dstp, half, ones, out, idx_v, ones_v, bounce, table, sem):
    c = lax.axis_index("c")
    s = lax.axis_index("s")
    w = s * NC + c
    rows = pl.ds(s * RPT, RPT)
    pltpu.sync_copy(half.at[rows], bounce)
    pltpu.sync_copy(bounce, table.at[rows])
    pltpu.sync_copy(ones, ones_v)
    pltpu.sync_copy(dstp.at[w], idx_v)
    plsc.subcore_barrier()

    def body(j, carry):
        pltpu.sync_copy(ones_v, table.at[idx_v.at[j]], add=True)
        return carry

    lax.fori_loop(0, CPW, body, 0)
    plsc.subcore_barrier()
    pltpu.sync_copy(table.at[rows], bounce)
    pltpu.sync_copy(bounce, out.at[c, rows])


def _make_agg(D, split, spmem_gather=False, dtype=jnp.float32):
    c0, c1 = split
    cmax = max(c0, c1)
    scratch = [
        pltpu.VMEM((cmax, CHUNK), jnp.int32),
        pltpu.VMEM((cmax, CHUNK), jnp.int32),
        pltpu.VMEM((CHUNK, D), dtype),
        pltpu.VMEM((CHUNK, D), dtype),
        pltpu.VMEM((RPT, D), dtype),
        pltpu.VMEM_SHARED((NP, D), dtype),
        pltpu.SemaphoreType.DMA,
        pltpu.SemaphoreType.DMA,
    ]
    if spmem_gather:
        scratch.insert(5, pltpu.VMEM_SHARED((NP, D), dtype))

    @functools.partial(
        pl.kernel,
        out_type=jax.ShapeDtypeStruct((NC, NP, D), dtype),
        mesh=_mesh,
        compiler_params=_sc_params,
        scratch_types=scratch,
    )
    def agg(gtab_hbm, ghalf, srcq, dstq, out, idxs_v, idxd_v, rows_a, rows_b,
            bounce, *rest):
        if spmem_gather:
            gsh, table, sema, semb = rest
        else:
            table, sema, semb = rest
        c = lax.axis_index("c")
        s = lax.axis_index("s")
        rows = pl.ds(s * RPT, RPT)
        cnt = jnp.where(c == 0, c0, c1)
        pltpu.sync_copy(ghalf.at[rows], bounce)
        pltpu.sync_copy(bounce, table.at[rows])
        if spmem_gather:
            pltpu.sync_copy(gtab_hbm.at[rows], bounce)
            pltpu.sync_copy(bounce, gsh.at[rows])
            gtab = gsh
        else:
            gtab = gtab_hbm
        pltpu.sync_copy(srcq.at[c, s], idxs_v)
        pltpu.sync_copy(dstq.at[c, s], idxd_v)
        plsc.subcore_barrier()

        pltpu.async_copy(gtab.at[idxs_v.at[0]], rows_a, sema)

        def body(k, carry):
            j = 2 * k
            pltpu.make_async_copy(gtab.at[idxs_v.at[j]], rows_a, sema).wait()
            pltpu.async_copy(gtab.at[idxs_v.at[j + 1]], rows_b, semb)
            pltpu.sync_copy(rows_a, table.at[idxd_v.at[j]], add=True)
            pltpu.make_async_copy(gtab.at[idxs_v.at[j + 1]], rows_b, semb).wait()

            @pl.when(j + 2 < cnt)
            def _():
                pltpu.async_copy(gtab.at[idxs_v.at[j + 2]], rows_a, sema)

            pltpu.sync_copy(rows_b, table.at[idxd_v.at[j + 1]], add=True)
            return carry

        lax.fori_loop(0, cnt // 2, body, 0)
        plsc.subcore_barrier()
        pltpu.sync_copy(table.at[rows], bounce)
        pltpu.sync_copy(bounce, out.at[c, rows])

    return agg


_agg64 = _make_agg(DH1, AGG64_SPLIT, spmem_gather=True, dtype=jnp.bfloat16)
_agg32 = _make_agg(DH2, AGG32_SPLIT, spmem_gather=True)


_G2C0, _G2C1 = G2_SPLIT
_G2MAX = max(_G2C0, _G2C1)


@functools.partial(
    pl.kernel,
    out_type=jax.ShapeDtypeStruct((TOTCH * 2 * CHUNK, DH2), jnp.float32),
    mesh=_mesh,
    compiler_params=_sc_params,
    scratch_types=[
        pltpu.VMEM((_G2MAX, CHUNK), jnp.int32),
        pltpu.VMEM((_G2MAX, CHUNK), jnp.int32),
        pltpu.VMEM((CHUNK, DH2), jnp.float32),
        pltpu.VMEM((CHUNK, DH2), jnp.float32),
        pltpu.VMEM((RPT, DH2), jnp.float32),
        pltpu.VMEM_SHARED((NP, DH2), jnp.float32),
        pltpu.SemaphoreType.DMA,
        pltpu.SemaphoreType.DMA,
    ],
)
def _gather2_kernel(zhbm, srcq, dstq, out, idx0_v, idx1_v, rows_a, rows_b,
                    bounce, z, sema, semb):
    c = lax.axis_index("c")
    s = lax.axis_index("s")
    rows = pl.ds(s * RPT, RPT)
    cnt = jnp.where(c == 0, _G2C0, _G2C1)
    goff = jnp.where(c == 0, s * _G2C0, NS * _G2C0 + s * _G2C1)
    pltpu.sync_copy(zhbm.at[rows], bounce)
    pltpu.sync_copy(bounce, z.at[rows])
    pltpu.sync_copy(srcq.at[c, s], idx0_v)
    pltpu.sync_copy(dstq.at[c, s], idx1_v)
    plsc.subcore_barrier()

    pltpu.async_copy(z.at[idx0_v.at[0]], rows_a, sema)

    def body(j, carry):
        g = goff + j
        pltpu.make_async_copy(z.at[idx0_v.at[j]], rows_a, sema).wait()
        pltpu.async_copy(z.at[idx1_v.at[j]], rows_b, semb)
        pltpu.sync_copy(rows_a, out.at[pl.ds(g * (2 * CHUNK), CHUNK)])
        pltpu.make_async_copy(z.at[idx1_v.at[j]], rows_b, semb).wait()

        @pl.when(j + 1 < cnt)
        def _():
            pltpu.async_copy(z.at[idx0_v.at[j + 1]], rows_a, sema)

        pltpu.sync_copy(rows_b, out.at[pl.ds(g * (2 * CHUNK) + CHUNK, CHUNK)])
        return carry

    lax.fori_loop(0, cnt, body, 0)


_BLK = 1024
_NB = NP // _BLK


def _tc1_body(degp, emb, w1, g1_o, dinv_o):
    deg = degp[0] + degp[1] + 1.0
    dinv = jnp.where(deg > 0, lax.rsqrt(jnp.maximum(deg, 1e-12)), 0.0)
    h = jnp.dot(emb[...], w1[...], preferred_element_type=jnp.float32)
    g = dinv[:, 0:1] * h
    g1_o[...] = g.astype(jnp.bfloat16)
    dinv_o[...] = dinv


_tc1 = pl.pallas_call(
    _tc1_body,
    grid=(_NB,),
    in_specs=[
        pl.BlockSpec((NC, _BLK, 8), lambda i: (0, i, 0)),
        pl.BlockSpec((_BLK, D_EMB), lambda i: (i, 0)),
        pl.BlockSpec((D_EMB, DH1), lambda i: (0, 0)),
    ],
    out_specs=[
        pl.BlockSpec((_BLK, DH1), lambda i: (i, 0)),
        pl.BlockSpec((_BLK, 8), lambda i: (i, 0)),
    ],
    out_shape=[
        jax.ShapeDtypeStruct((NP, DH1), jnp.bfloat16),
        jax.ShapeDtypeStruct((NP, 8), jnp.float32),
    ],
)


def _tc2_body(p, g1, dinv, b1, w2, g2_o):
    s = (p[0].astype(jnp.float32) + p[1].astype(jnp.float32)
         + g1[...].astype(jnp.float32))
    out1 = dinv[:, 0:1] * s + b1[...]
    x1 = jnp.maximum(out1, 0.0)
    h2 = jnp.dot(x1, w2[...], preferred_element_type=jnp.float32)
    g2_o[...] = dinv[:, 0:1] * h2


_tc2 = pl.pallas_call(
    _tc2_body,
    grid=(_NB,),
    in_specs=[
        pl.BlockSpec((NC, _BLK, DH1), lambda i: (0, i, 0)),
        pl.BlockSpec((_BLK, DH1), lambda i: (i, 0)),
        pl.BlockSpec((_BLK, 8), lambda i: (i, 0)),
        pl.BlockSpec((1, DH1), lambda i: (0, 0)),
        pl.BlockSpec((DH1, DH2), lambda i: (0, 0)),
    ],
    out_specs=pl.BlockSpec((_BLK, DH2), lambda i: (i, 0)),
    out_shape=jax.ShapeDtypeStruct((NP, DH2), jnp.float32),
)


def _tc3_body(q, g2, dinv, b2, z_o):
    s = q[0] + q[1] + g2[...]
    z_o[...] = dinv[:, 0:1] * s + b2[...]


_tc3 = pl.pallas_call(
    _tc3_body,
    grid=(_NB,),
    in_specs=[
        pl.BlockSpec((NC, _BLK, DH2), lambda i: (0, i, 0)),
        pl.BlockSpec((_BLK, DH2), lambda i: (i, 0)),
        pl.BlockSpec((_BLK, 8), lambda i: (i, 0)),
        pl.BlockSpec((1, DH2), lambda i: (0, 0)),
    ],
    out_specs=pl.BlockSpec((_BLK, DH2), lambda i: (i, 0)),
    out_shape=jax.ShapeDtypeStruct((NP, DH2), jnp.float32),
)


_GD = 64
_NDG = TOTCH // _GD


def _dec_body(a_ref, b_ref, p_o):
    a = a_ref[...]
    b = b_ref[...]
    ds = []
    for k in range(4):
        sl = slice(k * DH2, (k + 1) * DH2)
        ds.append(jnp.sum(a[:, 0, :, sl] * b[:, 0, :, sl], axis=2))
    d = jnp.concatenate(ds, axis=1)
    p_o[...] = 1.0 / (1.0 + jnp.exp(-d))


_tc_dec = pl.pallas_call(
    _dec_body,
    grid=(_NDG,),
    in_specs=[
        pl.BlockSpec((_GD, 1, 32, 128), lambda i: (i, 0, 0, 0)),
        pl.BlockSpec((_GD, 1, 32, 128), lambda i: (i, 1, 0, 0)),
    ],
    out_specs=pl.BlockSpec((_GD, 128), lambda i: (i, 0)),
    out_shape=jax.ShapeDtypeStruct((TOTCH, 128), jnp.float32),
)


def _split_chunks(flat, c0, c1):
    cmax = max(c0, c1)
    a = flat[: NS * c0].reshape(NS, c0, CHUNK)
    b = flat[NS * c0:].reshape(NS, c1, CHUNK)
    a = jnp.pad(a, ((0, 0), (0, cmax - c0), (0, 0)))
    b = jnp.pad(b, ((0, 0), (0, cmax - c1), (0, 0)))
    return jnp.stack([a, b])


def kernel(edge_index, emb, W1, b1, W2, b2):
    src = edge_index[0]
    dst = edge_index[1]
    pad = EP - E
    dpad = N + (jnp.arange(pad, dtype=jnp.int32) % (NP - N))
    srcf = jnp.concatenate([src, jnp.zeros((pad,), jnp.int32)]).reshape(TOTCH, CHUNK)
    dstf = jnp.concatenate([dst, dpad]).reshape(TOTCH, CHUNK)
    dstp = dstf.reshape(NW, CPW, CHUNK)
    src64 = _split_chunks(srcf, *AGG64_SPLIT)
    dst64 = _split_chunks(dstf, *AGG64_SPLIT)
    src32 = _split_chunks(srcf, *AGG32_SPLIT)
    dst32 = _split_chunks(dstf, *AGG32_SPLIT)
    eos = (32 * (jnp.arange(CHUNK) % 4) + jnp.arange(CHUNK) // 4)
    srcg2 = _split_chunks(srcf[:, eos], *G2_SPLIT)
    dstg2 = _split_chunks(dstf[:, eos], *G2_SPLIT)
    zero8 = jnp.zeros((NP, 8), jnp.float32)
    zero64 = jnp.zeros((NP, DH1), jnp.bfloat16)
    zero32 = jnp.zeros((NP, DH2), jnp.float32)
    ones8 = jnp.ones((CHUNK, 8), jnp.float32)
    emb_p = jnp.pad(emb, ((0, NP - N), (0, 0)))

    degp = _deg_kernel(dstp, zero8, ones8)
    g1, dinv8 = _tc1(degp, emb_p, W1)
    p1 = _agg64(g1, zero64, src64, dst64)
    g2 = _tc2(p1, g1, dinv8, b1.reshape(1, DH1), W2)
    p2 = _agg32(g2, zero32, src32, dst32)
    z = _tc3(p2, g2, dinv8, b2.reshape(1, DH2))
    zz = _gather2_kernel(z, srcg2, dstg2)
    probs = _tc_dec(zz.reshape(TOTCH, 2, 32, 128), zz.reshape(TOTCH, 2, 32, 128))
    return probs.reshape(EP)[:E]


# speedup vs baseline: 1.5564x; 1.0071x over previous
import functools

import jax
import jax.numpy as jnp
from jax import lax
from jax.experimental import pallas as pl
from jax.experimental.pallas import tpu as pltpu
from jax.experimental.pallas import tpu_sc as plsc

N = 10000
E = 320000
D_EMB = 128
DH1 = 64
DH2 = 32

NC = 2
NS = 16
NW = NC * NS
CHUNK = 128
CPW = 80
TOTCH = NW * CPW
EP = TOTCH * CHUNK
NP = 10240
RPT = NP // NS
DUMMY = N

AGG64_SPLIT = (88, 72)
AGG32_SPLIT = (88, 72)
G2_SPLIT = (84, 76)

_mesh = plsc.VectorSubcoreMesh(core_axis_name="c", subcore_axis_name="s")
_sc_params = pltpu.CompilerParams(use_tc_tiling_on_sc=False)


@functools.partial(
    pl.kernel,
    out_type=jax.ShapeDtypeStruct((NC, NP, 8), jnp.float32),
    mesh=_mesh,
    compiler_params=_sc_params,
    scratch_types=[
        pltpu.VMEM((CPW, CHUNK), jnp.int32),
        pltpu.VMEM((CHUNK, 8), jnp.float32),
        pltpu.VMEM((RPT, 8), jnp.float32),
        pltpu.VMEM_SHARED((NP, 8), jnp.float32),
        pltpu.SemaphoreType.DMA,
    ],
)
def _deg_kernel(dstp, half, ones, out, idx_v, ones_v, bounce, table, sem):
    c = lax.axis_index("c")
    s = lax.axis_index("s")
    w = s * NC + c
    rows = pl.ds(s * RPT, RPT)
    pltpu.sync_copy(half.at[rows], bounce)
    pltpu.sync_copy(bounce, table.at[rows])
    pltpu.sync_copy(ones, ones_v)
    pltpu.sync_copy(dstp.at[w], idx_v)
    plsc.subcore_barrier()

    def body(j, carry):
        pltpu.sync_copy(ones_v, table.at[idx_v.at[j]], add=True)
        return carry

    lax.fori_loop(0, CPW, body, 0)
    plsc.subcore_barrier()
    pltpu.sync_copy(table.at[rows], bounce)
    pltpu.sync_copy(bounce, out.at[c, rows])


def _make_agg(D, split, spmem_gather=False, dtype=jnp.float32):
    c0, c1 = split
    cmax = max(c0, c1)
    scratch = [
        pltpu.VMEM((cmax, CHUNK), jnp.int32),
        pltpu.VMEM((cmax, CHUNK), jnp.int32),
        pltpu.VMEM((CHUNK, D), dtype),
        pltpu.VMEM((CHUNK, D), dtype),
        pltpu.VMEM((RPT, D), dtype),
        pltpu.VMEM_SHARED((NP, D), dtype),
        pltpu.SemaphoreType.DMA,
        pltpu.SemaphoreType.DMA,
    ]
    if spmem_gather:
        scratch.insert(5, pltpu.VMEM_SHARED((NP, D), dtype))

    @functools.partial(
        pl.kernel,
        out_type=jax.ShapeDtypeStruct((NC, NP, D), dtype),
        mesh=_mesh,
        compiler_params=_sc_params,
        scratch_types=scratch,
    )
    def agg(gtab_hbm, ghalf, srcq, dstq, out, idxs_v, idxd_v, rows_a, rows_b,
            bounce, *rest):
        if spmem_gather:
            gsh, table, sema, semb = rest
        else:
            table, sema, semb = rest
        c = lax.axis_index("c")
        s = lax.axis_index("s")
        rows = pl.ds(s * RPT, RPT)
        cnt = jnp.where(c == 0, c0, c1)
        pltpu.sync_copy(ghalf.at[rows], bounce)
        pltpu.sync_copy(bounce, table.at[rows])
        if spmem_gather:
            pltpu.sync_copy(gtab_hbm.at[rows], bounce)
            pltpu.sync_copy(bounce, gsh.at[rows])
            gtab = gsh
        else:
            gtab = gtab_hbm
        pltpu.sync_copy(srcq.at[c, s], idxs_v)
        pltpu.sync_copy(dstq.at[c, s], idxd_v)
        plsc.subcore_barrier()

        pltpu.async_copy(gtab.at[idxs_v.at[0]], rows_a, sema)

        def body(k, carry):
            j = 2 * k
            pltpu.make_async_copy(gtab.at[idxs_v.at[j]], rows_a, sema).wait()
            pltpu.async_copy(gtab.at[idxs_v.at[j + 1]], rows_b, semb)
            pltpu.sync_copy(rows_a, table.at[idxd_v.at[j]], add=True)
            pltpu.make_async_copy(gtab.at[idxs_v.at[j + 1]], rows_b, semb).wait()

            @pl.when(j + 2 < cnt)
            def _():
                pltpu.async_copy(gtab.at[idxs_v.at[j + 2]], rows_a, sema)

            pltpu.sync_copy(rows_b, table.at[idxd_v.at[j + 1]], add=True)
            return carry

        lax.fori_loop(0, cnt // 2, body, 0)
        plsc.subcore_barrier()
        pltpu.sync_copy(table.at[rows], bounce)
        pltpu.sync_copy(bounce, out.at[c, rows])

    return agg


_agg64 = _make_agg(DH1, AGG64_SPLIT, spmem_gather=True, dtype=jnp.bfloat16)
_agg32 = _make_agg(DH2, AGG32_SPLIT, spmem_gather=True)


_G2C0, _G2C1 = G2_SPLIT
_G2MAX = max(_G2C0, _G2C1)


@functools.partial(
    pl.kernel,
    out_type=jax.ShapeDtypeStruct((TOTCH * 2 * CHUNK, DH2), jnp.float32),
    mesh=_mesh,
    compiler_params=_sc_params,
    scratch_types=[
        pltpu.VMEM((_G2MAX, CHUNK), jnp.int32),
        pltpu.VMEM((_G2MAX, CHUNK), jnp.int32),
        pltpu.VMEM((CHUNK, DH2), jnp.float32),
        pltpu.VMEM((CHUNK, DH2), jnp.float32),
        pltpu.VMEM((RPT, DH2), jnp.float32),
        pltpu.VMEM_SHARED((NP, DH2), jnp.float32),
        pltpu.SemaphoreType.DMA,
        pltpu.SemaphoreType.DMA,
    ],
)
def _gather2_kernel(zhbm, srcq, dstq, out, idx0_v, idx1_v, rows_a, rows_b,
                    bounce, z, sema, semb):
    c = lax.axis_index("c")
    s = lax.axis_index("s")
    rows = pl.ds(s * RPT, RPT)
    cnt = jnp.where(c == 0, _G2C0, _G2C1)
    goff = jnp.where(c == 0, s * _G2C0, NS * _G2C0 + s * _G2C1)
    pltpu.sync_copy(zhbm.at[rows], bounce)
    pltpu.sync_copy(bounce, z.at[rows])
    pltpu.sync_copy(srcq.at[c, s], idx0_v)
    pltpu.sync_copy(dstq.at[c, s], idx1_v)
    plsc.subcore_barrier()

    pltpu.async_copy(z.at[idx0_v.at[0]], rows_a, sema)

    def body(j, carry):
        g = goff + j
        pltpu.make_async_copy(z.at[idx0_v.at[j]], rows_a, sema).wait()
        pltpu.async_copy(z.at[idx1_v.at[j]], rows_b, semb)
        pltpu.sync_copy(rows_a, out.at[pl.ds(g * (2 * CHUNK), CHUNK)])
        pltpu.make_async_copy(z.at[idx1_v.at[j]], rows_b, semb).wait()

        @pl.when(j + 1 < cnt)
        def _():
            pltpu.async_copy(z.at[idx0_v.at[j + 1]], rows_a, sema)

        pltpu.sync_copy(rows_b, out.at[pl.ds(g * (2 * CHUNK) + CHUNK, CHUNK)])
        return carry

    lax.fori_loop(0, cnt, body, 0)


_BLK = 1024
_NB = NP // _BLK


def _tc1_body(degp, emb, w1, g1_o, dinv_o):
    deg = degp[0] + degp[1] + 1.0
    dinv = jnp.where(deg > 0, lax.rsqrt(jnp.maximum(deg, 1e-12)), 0.0)
    h = jnp.dot(emb[...], w1[...], preferred_element_type=jnp.float32)
    g = dinv[:, 0:1] * h
    g1_o[...] = g.astype(jnp.bfloat16)
    dinv_o[...] = dinv


_tc1 = pl.pallas_call(
    _tc1_body,
    grid=(_NB,),
    in_specs=[
        pl.BlockSpec((NC, _BLK, 8), lambda i: (0, i, 0)),
        pl.BlockSpec((_BLK, D_EMB), lambda i: (i, 0)),
        pl.BlockSpec((D_EMB, DH1), lambda i: (0, 0)),
    ],
    out_specs=[
        pl.BlockSpec((_BLK, DH1), lambda i: (i, 0)),
        pl.BlockSpec((_BLK, 8), lambda i: (i, 0)),
    ],
    out_shape=[
        jax.ShapeDtypeStruct((NP, DH1), jnp.bfloat16),
        jax.ShapeDtypeStruct((NP, 8), jnp.float32),
    ],
)


def _tc2_body(p, g1, dinv, b1, w2, g2_o):
    s = (p[0].astype(jnp.float32) + p[1].astype(jnp.float32)
         + g1[...].astype(jnp.float32))
    out1 = dinv[:, 0:1] * s + b1[...]
    x1 = jnp.maximum(out1, 0.0)
    h2 = jnp.dot(x1, w2[...], preferred_element_type=jnp.float32)
    g2_o[...] = dinv[:, 0:1] * h2


_tc2 = pl.pallas_call(
    _tc2_body,
    grid=(_NB,),
    in_specs=[
        pl.BlockSpec((NC, _BLK, DH1), lambda i: (0, i, 0)),
        pl.BlockSpec((_BLK, DH1), lambda i: (i, 0)),
        pl.BlockSpec((_BLK, 8), lambda i: (i, 0)),
        pl.BlockSpec((1, DH1), lambda i: (0, 0)),
        pl.BlockSpec((DH1, DH2), lambda i: (0, 0)),
    ],
    out_specs=pl.BlockSpec((_BLK, DH2), lambda i: (i, 0)),
    out_shape=jax.ShapeDtypeStruct((NP, DH2), jnp.float32),
)


def _tc3_body(q, g2, dinv, b2, z_o):
    s = q[0] + q[1] + g2[...]
    z_o[...] = dinv[:, 0:1] * s + b2[...]


_tc3 = pl.pallas_call(
    _tc3_body,
    grid=(_NB,),
    in_specs=[
        pl.BlockSpec((NC, _BLK, DH2), lambda i: (0, i, 0)),
        pl.BlockSpec((_BLK, DH2), lambda i: (i, 0)),
        pl.BlockSpec((_BLK, 8), lambda i: (i, 0)),
        pl.BlockSpec((1, DH2), lambda i: (0, 0)),
    ],
    out_specs=pl.BlockSpec((_BLK, DH2), lambda i: (i, 0)),
    out_shape=jax.ShapeDtypeStruct((NP, DH2), jnp.float32),
)


_GD = 128
_NDG = TOTCH // _GD


def _dec_body(a_ref, b_ref, p_o):
    a = a_ref[...]
    b = b_ref[...]
    ds = []
    for k in range(4):
        sl = slice(k * DH2, (k + 1) * DH2)
        ds.append(jnp.sum(a[:, 0, :, sl] * b[:, 0, :, sl], axis=2))
    d = jnp.concatenate(ds, axis=1)
    p_o[...] = 1.0 / (1.0 + jnp.exp(-d))


_tc_dec = pl.pallas_call(
    _dec_body,
    grid=(_NDG,),
    in_specs=[
        pl.BlockSpec((_GD, 1, 32, 128), lambda i: (i, 0, 0, 0)),
        pl.BlockSpec((_GD, 1, 32, 128), lambda i: (i, 1, 0, 0)),
    ],
    out_specs=pl.BlockSpec((_GD, 128), lambda i: (i, 0)),
    out_shape=jax.ShapeDtypeStruct((TOTCH, 128), jnp.float32),
)


def _split_chunks(flat, c0, c1):
    cmax = max(c0, c1)
    a = flat[: NS * c0].reshape(NS, c0, CHUNK)
    b = flat[NS * c0:].reshape(NS, c1, CHUNK)
    a = jnp.pad(a, ((0, 0), (0, cmax - c0), (0, 0)))
    b = jnp.pad(b, ((0, 0), (0, cmax - c1), (0, 0)))
    return jnp.stack([a, b])


def kernel(edge_index, emb, W1, b1, W2, b2):
    src = edge_index[0]
    dst = edge_index[1]
    pad = EP - E
    dpad = N + (jnp.arange(pad, dtype=jnp.int32) % (NP - N))
    srcf = jnp.concatenate([src, jnp.zeros((pad,), jnp.int32)]).reshape(TOTCH, CHUNK)
    dstf = jnp.concatenate([dst, dpad]).reshape(TOTCH, CHUNK)
    dstp = dstf.reshape(NW, CPW, CHUNK)
    src64 = _split_chunks(srcf, *AGG64_SPLIT)
    dst64 = _split_chunks(dstf, *AGG64_SPLIT)
    src32 = _split_chunks(srcf, *AGG32_SPLIT)
    dst32 = _split_chunks(dstf, *AGG32_SPLIT)
    eos = (32 * (jnp.arange(CHUNK) % 4) + jnp.arange(CHUNK) // 4)
    srcg2 = _split_chunks(srcf[:, eos], *G2_SPLIT)
    dstg2 = _split_chunks(dstf[:, eos], *G2_SPLIT)
    zero8 = jnp.zeros((NP, 8), jnp.float32)
    zero64 = jnp.zeros((NP, DH1), jnp.bfloat16)
    zero32 = jnp.zeros((NP, DH2), jnp.float32)
    ones8 = jnp.ones((CHUNK, 8), jnp.float32)
    emb_p = jnp.pad(emb, ((0, NP - N), (0, 0)))

    degp = _deg_kernel(dstp, zero8, ones8)
    g1, dinv8 = _tc1(degp, emb_p, W1)
    p1 = _agg64(g1, zero64, src64, dst64)
    g2 = _tc2(p1, g1, dinv8, b1.reshape(1, DH1), W2)
    p2 = _agg32(g2, zero32, src32, dst32)
    z = _tc3(p2, g2, dinv8, b2.reshape(1, DH2))
    zz = _gather2_kernel(z, srcg2, dstg2)
    probs = _tc_dec(zz.reshape(TOTCH, 2, 32, 128), zz.reshape(TOTCH, 2, 32, 128))
    return probs.reshape(EP)[:E]
